# Initial kernel scaffold; baseline (speedup 1.0000x reference)
import jax
import jax.numpy as jnp
from jax.experimental import pallas as pl


def kernel(feat, edge_index):
    raise NotImplementedError("write your pallas kernel here")


# speedup vs baseline: 1.4367x; 1.4367x over previous
import functools

import jax
import jax.numpy as jnp
from jax import lax
from jax.experimental import pallas as pl
from jax.experimental.pallas import tpu as pltpu
from jax.experimental.pallas import tpu_sc as plsc

_LANES = 16
_CHUNK = 64

_DNUMS = lax.GatherDimensionNumbers(
    offset_dims=(), collapsed_slice_dims=(0,), start_index_map=(0,))


def _xlane(v, idx):
    return lax.gather(v, idx[:, None], dimension_numbers=_DNUMS,
                      slice_sizes=(1,),
                      mode=lax.GatherScatterMode.PROMISE_IN_BOUNDS)


@functools.lru_cache(maxsize=None)
def _make_sc_kernel(n_nodes, d_feat, n_edges_pad):
    info = plsc.get_sparse_core_info()
    nc, ns = info.num_cores, info.num_subcores
    nw = nc * ns
    assert n_edges_pad % (nw * _CHUNK) == 0
    epw = n_edges_pad // nw
    nch = epw // _CHUNK
    n_seg = d_feat // _LANES
    assert d_feat % (2 * _LANES) == 0

    mesh = plsc.VectorSubcoreMesh(core_axis_name="c", subcore_axis_name="s")

    @functools.partial(
        pl.kernel,
        mesh=mesh,
        out_type=jax.ShapeDtypeStruct((n_edges_pad,), jnp.float32),
        scratch_types=[
            pltpu.VMEM((epw,), jnp.int32),
            pltpu.VMEM((epw,), jnp.int32),
            pltpu.VMEM((_CHUNK, d_feat), jnp.float32),
            pltpu.VMEM((_CHUNK, d_feat), jnp.float32),
            pltpu.VMEM((epw,), jnp.float32),
            pltpu.SemaphoreType.DMA,
            pltpu.SemaphoreType.DMA,
        ],
    )
    def k(feat_hbm, src_hbm, dst_hbm, out_hbm,
          src_v, dst_v, u_b, v_b, out_v, sem_u, sem_v):
        wid = lax.axis_index("s") * nc + lax.axis_index("c")
        base = wid * epw
        pltpu.sync_copy(src_hbm.at[pl.ds(base, epw)], src_v)
        pltpu.sync_copy(dst_hbm.at[pl.ds(base, epw)], dst_v)
        lane = lax.iota(jnp.int32, _LANES)

        def chunk_body(ch, _):
            off = pl.multiple_of(ch * _CHUNK, 8)
            cu = pltpu.make_async_copy(
                feat_hbm.at[src_v.at[pl.ds(off, _CHUNK)]], u_b, sem_u)
            cv = pltpu.make_async_copy(
                feat_hbm.at[dst_v.at[pl.ds(off, _CHUNK)]], v_b, sem_v)
            cu.start()
            cv.start()
            cu.wait()
            cv.wait()

            def group_body(g, _):
                eb = g * _LANES
                res = jnp.zeros((_LANES,), jnp.float32)
                for el in range(_LANES):
                    e = eb + el
                    acc0 = u_b[e, pl.ds(0, _LANES)] * v_b[e, pl.ds(0, _LANES)]
                    acc1 = (u_b[e, pl.ds(_LANES, _LANES)]
                            * v_b[e, pl.ds(_LANES, _LANES)])
                    for t in range(2, n_seg, 2):
                        acc0 = acc0 + (u_b[e, pl.ds(t * _LANES, _LANES)]
                                       * v_b[e, pl.ds(t * _LANES, _LANES)])
                        acc1 = acc1 + (u_b[e, pl.ds((t + 1) * _LANES, _LANES)]
                                       * v_b[e, pl.ds((t + 1) * _LANES, _LANES)])
                    acc = acc0 + acc1
                    for hop in (8, 4, 2, 1):
                        acc = acc + _xlane(acc, lane ^ hop)
                    res = jnp.where(lane == el, acc, res)
                out_v[pl.ds(off + eb, _LANES)] = res
                return 0

            lax.fori_loop(0, _CHUNK // _LANES, group_body, 0)
            return 0

        lax.fori_loop(0, nch, chunk_body, 0)
        pltpu.sync_copy(out_v, out_hbm.at[pl.ds(base, epw)])

    return k


def kernel(feat, edge_index):
    n_nodes, d_feat = feat.shape
    n_edges = edge_index.shape[1]
    ei = edge_index.astype(jnp.int32)
    pad = (-n_edges) % (32 * _CHUNK)
    src = jnp.pad(ei[0], (0, pad))
    dst = jnp.pad(ei[1], (0, pad))
    k = _make_sc_kernel(n_nodes, d_feat, n_edges + pad)
    score = k(feat, src, dst)
    return score[:n_edges].reshape(n_edges, 1)


# speedup vs baseline: 1.6764x; 1.1669x over previous
import functools

import jax
import jax.numpy as jnp
from jax import lax
from jax.experimental import pallas as pl
from jax.experimental.pallas import tpu as pltpu
from jax.experimental.pallas import tpu_sc as plsc

_LANES = 16
_CHUNK = 64

_DNUMS = lax.GatherDimensionNumbers(
    offset_dims=(), collapsed_slice_dims=(0,), start_index_map=(0,))


def _xlane(v, idx):
    return lax.gather(v, idx[:, None], dimension_numbers=_DNUMS,
                      slice_sizes=(1,),
                      mode=lax.GatherScatterMode.PROMISE_IN_BOUNDS)


@functools.lru_cache(maxsize=None)
def _make_sc_kernel(n_nodes, d_feat, n_edges_pad):
    info = plsc.get_sparse_core_info()
    nc, ns = info.num_cores, info.num_subcores
    nw = nc * ns
    assert n_edges_pad % (nw * _CHUNK) == 0
    epw = n_edges_pad // nw
    nch = epw // _CHUNK
    assert nch % 2 == 0
    n_seg = d_feat // _LANES
    assert d_feat % (2 * _LANES) == 0

    mesh = plsc.VectorSubcoreMesh(core_axis_name="c", subcore_axis_name="s")

    @functools.partial(
        pl.kernel,
        mesh=mesh,
        out_type=jax.ShapeDtypeStruct((n_edges_pad,), jnp.float32),
        scratch_types=[
            pltpu.VMEM((epw,), jnp.int32),
            pltpu.VMEM((epw,), jnp.int32),
            pltpu.VMEM((_CHUNK, d_feat), jnp.float32),
            pltpu.VMEM((_CHUNK, d_feat), jnp.float32),
            pltpu.VMEM((_CHUNK, d_feat), jnp.float32),
            pltpu.VMEM((_CHUNK, d_feat), jnp.float32),
            pltpu.VMEM((epw,), jnp.float32),
            pltpu.SemaphoreType.DMA,
            pltpu.SemaphoreType.DMA,
        ],
    )
    def k(feat_hbm, src_hbm, dst_hbm, out_hbm,
          src_v, dst_v, u_a, v_a, u_b, v_b, out_v, sem_a, sem_b):
        wid = lax.axis_index("s") * nc + lax.axis_index("c")
        base = wid * epw
        pltpu.sync_copy(src_hbm.at[pl.ds(base, epw)], src_v)
        pltpu.sync_copy(dst_hbm.at[pl.ds(base, epw)], dst_v)
        lane = lax.iota(jnp.int32, _LANES)

        def copies(ch, ub, vb, sem):
            off = pl.multiple_of(ch * _CHUNK, 8)
            cu = pltpu.make_async_copy(
                feat_hbm.at[src_v.at[pl.ds(off, _CHUNK)]], ub, sem)
            cv = pltpu.make_async_copy(
                feat_hbm.at[dst_v.at[pl.ds(off, _CHUNK)]], vb, sem)
            return cu, cv

        def fire(ch, ub, vb, sem):
            cu, cv = copies(ch, ub, vb, sem)
            cu.start()
            cv.start()

        def wait(ch, ub, vb, sem):
            cu, cv = copies(ch, ub, vb, sem)
            cu.wait()
            cv.wait()

        def compute(ch, ub, vb):
            off = pl.multiple_of(ch * _CHUNK, 8)

            def edge_dot(e):
                acc0 = ub[e, pl.ds(0, _LANES)] * vb[e, pl.ds(0, _LANES)]
                acc1 = (ub[e, pl.ds(_LANES, _LANES)]
                        * vb[e, pl.ds(_LANES, _LANES)])
                for t in range(2, n_seg, 2):
                    acc0 = acc0 + (ub[e, pl.ds(t * _LANES, _LANES)]
                                   * vb[e, pl.ds(t * _LANES, _LANES)])
                    acc1 = acc1 + (ub[e, pl.ds((t + 1) * _LANES, _LANES)]
                                   * vb[e, pl.ds((t + 1) * _LANES, _LANES)])
                acc = acc0 + acc1
                for hop in (8, 4, 2, 1):
                    acc = acc + _xlane(acc, lane ^ hop)
                return acc

            def group_body(g, _):
                eb = g * _LANES

                def pair_body(j, res):
                    e = eb + 2 * j
                    res = jnp.where(lane == 2 * j, edge_dot(e), res)
                    res = jnp.where(lane == 2 * j + 1, edge_dot(e + 1), res)
                    return res

                res = lax.fori_loop(0, _LANES // 2, pair_body,
                                    jnp.zeros((_LANES,), jnp.float32))
                out_v[pl.ds(off + eb, _LANES)] = res
                return 0

            lax.fori_loop(0, _CHUNK // _LANES, group_body, 0)

        fire(0, u_a, v_a, sem_a)
        fire(1, u_b, v_b, sem_b)

        def pair_chunks(i, _):
            ch0 = 2 * i
            wait(ch0, u_a, v_a, sem_a)
            compute(ch0, u_a, v_a)

            @pl.when(i < nch // 2 - 1)
            def _():
                fire(ch0 + 2, u_a, v_a, sem_a)

            ch1 = 2 * i + 1
            wait(ch1, u_b, v_b, sem_b)
            compute(ch1, u_b, v_b)

            @pl.when(i < nch // 2 - 1)
            def _():
                fire(ch1 + 2, u_b, v_b, sem_b)

            return 0

        lax.fori_loop(0, nch // 2, pair_chunks, 0)
        pltpu.sync_copy(out_v, out_hbm.at[pl.ds(base, epw)])

    return k


def kernel(feat, edge_index):
    n_nodes, d_feat = feat.shape
    n_edges = edge_index.shape[1]
    ei = edge_index.astype(jnp.int32)
    pad = (-n_edges) % (32 * _CHUNK * 2)
    src = jnp.pad(ei[0], (0, pad))
    dst = jnp.pad(ei[1], (0, pad))
    k = _make_sc_kernel(n_nodes, d_feat, n_edges + pad)
    score = k(feat, src, dst)
    return score[:n_edges].reshape(n_edges, 1)


# speedup vs baseline: 4.7236x; 2.8177x over previous
import functools

import jax
import jax.numpy as jnp
from jax import lax
from jax.experimental import pallas as pl
from jax.experimental.pallas import tpu as pltpu
from jax.experimental.pallas import tpu_sc as plsc

_LANES = 16
_CHUNK = 64

_DNUMS = lax.GatherDimensionNumbers(
    offset_dims=(), collapsed_slice_dims=(0,), start_index_map=(0,))


def _xlane(v, idx):
    return lax.gather(v, idx[:, None], dimension_numbers=_DNUMS,
                      slice_sizes=(1,),
                      mode=lax.GatherScatterMode.PROMISE_IN_BOUNDS)


@functools.lru_cache(maxsize=None)
def _make_sc_kernel(n_nodes, d_feat, n_edges_pad):
    info = plsc.get_sparse_core_info()
    nc, ns = info.num_cores, info.num_subcores
    nw = nc * ns
    assert n_edges_pad % (nw * _CHUNK) == 0
    epw = n_edges_pad // nw
    nch = epw // _CHUNK
    assert nch % 2 == 0
    assert d_feat % (4 * _LANES) == 0
    d_half = d_feat // 2
    assert n_nodes % (16 * ns) == 0
    rows_per_sub = n_nodes // ns

    mesh = plsc.VectorSubcoreMesh(core_axis_name="c", subcore_axis_name="s")

    @functools.partial(
        pl.kernel,
        mesh=mesh,
        out_type=jax.ShapeDtypeStruct((n_edges_pad,), jnp.float32),
        scratch_types=[
            pltpu.VMEM_SHARED((n_nodes, d_half), jnp.int32),
            pltpu.VMEM((epw,), jnp.int32),
            pltpu.VMEM((epw,), jnp.int32),
            pltpu.VMEM((_CHUNK, d_half), jnp.int32),
            pltpu.VMEM((_CHUNK, d_half), jnp.int32),
            pltpu.VMEM((_CHUNK, d_half), jnp.int32),
            pltpu.VMEM((_CHUNK, d_half), jnp.int32),
            pltpu.VMEM((epw,), jnp.float32),
            pltpu.SemaphoreType.DMA,
            pltpu.SemaphoreType.DMA,
        ],
    )
    def k(feat_hbm, src_hbm, dst_hbm, out_hbm,
          table, src_v, dst_v, u_a, v_a, u_b, v_b, out_v, sem_a, sem_b):
        sid = lax.axis_index("s")
        wid = sid * nc + lax.axis_index("c")
        base = wid * epw
        row0 = sid * rows_per_sub
        pltpu.sync_copy(feat_hbm.at[pl.ds(row0, rows_per_sub)],
                        table.at[pl.ds(row0, rows_per_sub)])
        pltpu.sync_copy(src_hbm.at[pl.ds(base, epw)], src_v)
        pltpu.sync_copy(dst_hbm.at[pl.ds(base, epw)], dst_v)
        plsc.subcore_barrier()
        lane = lax.iota(jnp.int32, _LANES)

        def copies(ch, ub, vb, sem):
            off = pl.multiple_of(ch * _CHUNK, 8)
            cu = pltpu.make_async_copy(
                table.at[src_v.at[pl.ds(off, _CHUNK)]], ub, sem)
            cv = pltpu.make_async_copy(
                table.at[dst_v.at[pl.ds(off, _CHUNK)]], vb, sem)
            return cu, cv

        def fire(ch, ub, vb, sem):
            cu, cv = copies(ch, ub, vb, sem)
            cu.start()
            cv.start()

        def wait(ch, ub, vb, sem):
            cu, cv = copies(ch, ub, vb, sem)
            cu.wait()
            cv.wait()

        def compute(ch, ub, vb):
            off = pl.multiple_of(ch * _CHUNK, 8)

            hi_mask = jnp.full((_LANES,), -0x10000, jnp.int32)

            def widen(wi):
                even = lax.bitcast_convert_type(wi << 16, jnp.float32)
                odd = lax.bitcast_convert_type(wi & hi_mask, jnp.float32)
                return even, odd

            def edge_dot(e):
                acc0 = jnp.zeros((_LANES,), jnp.float32)
                acc1 = jnp.zeros((_LANES,), jnp.float32)
                for t in range(d_half // _LANES):
                    uw = ub[e, pl.ds(t * _LANES, _LANES)]
                    vw = vb[e, pl.ds(t * _LANES, _LANES)]
                    u0, u1 = widen(uw)
                    v0, v1 = widen(vw)
                    acc0 = acc0 + u0 * v0
                    acc1 = acc1 + u1 * v1
                acc = acc0 + acc1
                for hop in (8, 4, 2, 1):
                    acc = acc + _xlane(acc, lane ^ hop)
                return acc

            def group_body(g, _):
                eb = g * _LANES

                def pair_body(j, res):
                    e = eb + 2 * j
                    res = jnp.where(lane == 2 * j, edge_dot(e), res)
                    res = jnp.where(lane == 2 * j + 1, edge_dot(e + 1), res)
                    return res

                res = lax.fori_loop(0, _LANES // 2, pair_body,
                                    jnp.zeros((_LANES,), jnp.float32))
                out_v[pl.ds(off + eb, _LANES)] = res
                return 0

            lax.fori_loop(0, _CHUNK // _LANES, group_body, 0)

        fire(0, u_a, v_a, sem_a)
        fire(1, u_b, v_b, sem_b)

        def pair_chunks(i, _):
            ch0 = 2 * i
            wait(ch0, u_a, v_a, sem_a)
            compute(ch0, u_a, v_a)

            @pl.when(i < nch // 2 - 1)
            def _():
                fire(ch0 + 2, u_a, v_a, sem_a)

            ch1 = 2 * i + 1
            wait(ch1, u_b, v_b, sem_b)
            compute(ch1, u_b, v_b)

            @pl.when(i < nch // 2 - 1)
            def _():
                fire(ch1 + 2, u_b, v_b, sem_b)

            return 0

        lax.fori_loop(0, nch // 2, pair_chunks, 0)
        pltpu.sync_copy(out_v, out_hbm.at[pl.ds(base, epw)])

    return k


def kernel(feat, edge_index):
    n_nodes, d_feat = feat.shape
    n_edges = edge_index.shape[1]
    ei = edge_index.astype(jnp.int32)
    pad = (-n_edges) % (32 * _CHUNK * 2)
    src = jnp.pad(ei[0], (0, pad))
    dst = jnp.pad(ei[1], (0, pad))
    node_pad = (-n_nodes) % 256
    feat_bf = jnp.pad(feat.astype(jnp.bfloat16), ((0, node_pad), (0, 0)))
    feat_packed = jax.lax.bitcast_convert_type(
        feat_bf.reshape(n_nodes + node_pad, d_feat // 2, 2), jnp.int32)
    k = _make_sc_kernel(n_nodes + node_pad, d_feat, n_edges + pad)
    score = k(feat_packed, src, dst)
    return score[:n_edges].reshape(n_edges, 1)


# speedup vs baseline: 5.3462x; 1.1318x over previous
import functools

import jax
import jax.numpy as jnp
from jax import lax
from jax.experimental import pallas as pl
from jax.experimental.pallas import tpu as pltpu
from jax.experimental.pallas import tpu_sc as plsc

_LANES = 16
_CHUNK = 32

_DNUMS = lax.GatherDimensionNumbers(
    offset_dims=(), collapsed_slice_dims=(0,), start_index_map=(0,))


def _xlane(v, idx):
    return lax.gather(v, idx[:, None], dimension_numbers=_DNUMS,
                      slice_sizes=(1,),
                      mode=lax.GatherScatterMode.PROMISE_IN_BOUNDS)


@functools.lru_cache(maxsize=None)
def _make_sc_kernel(n_nodes, d_feat, n_edges_pad):
    info = plsc.get_sparse_core_info()
    nc, ns = info.num_cores, info.num_subcores
    nw = nc * ns
    assert n_edges_pad % (nw * _CHUNK) == 0
    epw = n_edges_pad // nw
    nch = epw // _CHUNK
    assert nch % 4 == 0
    assert d_feat % (4 * _LANES) == 0
    d_half = d_feat // 2
    assert n_nodes % (16 * ns) == 0
    rows_per_sub = n_nodes // ns

    mesh = plsc.VectorSubcoreMesh(core_axis_name="c", subcore_axis_name="s")

    @functools.partial(
        pl.kernel,
        mesh=mesh,
        out_type=jax.ShapeDtypeStruct((n_edges_pad,), jnp.float32),
        scratch_types=[
            pltpu.VMEM_SHARED((n_nodes, d_half), jnp.int32),
            pltpu.VMEM((epw,), jnp.int32),
            pltpu.VMEM((epw,), jnp.int32),
            pltpu.VMEM((_CHUNK, d_half), jnp.int32),
            pltpu.VMEM((_CHUNK, d_half), jnp.int32),
            pltpu.VMEM((_CHUNK, d_half), jnp.int32),
            pltpu.VMEM((_CHUNK, d_half), jnp.int32),
            pltpu.VMEM((_CHUNK, d_half), jnp.int32),
            pltpu.VMEM((_CHUNK, d_half), jnp.int32),
            pltpu.VMEM((_CHUNK, d_half), jnp.int32),
            pltpu.VMEM((_CHUNK, d_half), jnp.int32),
            pltpu.VMEM((epw,), jnp.float32),
            pltpu.SemaphoreType.DMA,
            pltpu.SemaphoreType.DMA,
            pltpu.SemaphoreType.DMA,
            pltpu.SemaphoreType.DMA,
        ],
    )
    def k(feat_hbm, src_hbm, dst_hbm, out_hbm,
          table, src_v, dst_v, u_0, v_0, u_1, v_1, u_2, v_2, u_3, v_3,
          out_v, sem_0, sem_1, sem_2, sem_3):
        sid = lax.axis_index("s")
        wid = sid * nc + lax.axis_index("c")
        base = wid * epw
        row0 = sid * rows_per_sub
        pltpu.sync_copy(feat_hbm.at[pl.ds(row0, rows_per_sub)],
                        table.at[pl.ds(row0, rows_per_sub)])
        pltpu.sync_copy(src_hbm.at[pl.ds(base, epw)], src_v)
        pltpu.sync_copy(dst_hbm.at[pl.ds(base, epw)], dst_v)
        plsc.subcore_barrier()
        lane = lax.iota(jnp.int32, _LANES)

        def copies(ch, ub, vb, sem):
            off = pl.multiple_of(ch * _CHUNK, 8)
            cu = pltpu.make_async_copy(
                table.at[src_v.at[pl.ds(off, _CHUNK)]], ub, sem)
            cv = pltpu.make_async_copy(
                table.at[dst_v.at[pl.ds(off, _CHUNK)]], vb, sem)
            return cu, cv

        def fire(ch, ub, vb, sem):
            cu, cv = copies(ch, ub, vb, sem)
            cu.start()
            cv.start()

        def wait(ch, ub, vb, sem):
            cu, cv = copies(ch, ub, vb, sem)
            cu.wait()
            cv.wait()

        def compute(ch, ub, vb):
            off = pl.multiple_of(ch * _CHUNK, 8)

            hi_mask = jnp.full((_LANES,), -0x10000, jnp.int32)

            def widen(wi):
                even = lax.bitcast_convert_type(wi << 16, jnp.float32)
                odd = lax.bitcast_convert_type(wi, jnp.float32)
                return even, odd

            def edge_dot(e):
                acc0 = jnp.zeros((_LANES,), jnp.float32)
                acc1 = jnp.zeros((_LANES,), jnp.float32)
                for t in range(d_half // _LANES):
                    uw = ub[e, pl.ds(t * _LANES, _LANES)]
                    vw = vb[e, pl.ds(t * _LANES, _LANES)]
                    u0, u1 = widen(uw)
                    v0, v1 = widen(vw)
                    acc0 = acc0 + u0 * v0
                    acc1 = acc1 + u1 * v1
                acc = acc0 + acc1
                for hop in (8, 4, 2, 1):
                    acc = acc + _xlane(acc, lane ^ hop)
                return acc

            def group_body(g, _):
                eb = g * _LANES

                def pair_body(j, res):
                    e = eb + 2 * j
                    res = jnp.where(lane == 2 * j, edge_dot(e), res)
                    res = jnp.where(lane == 2 * j + 1, edge_dot(e + 1), res)
                    return res

                res = lax.fori_loop(0, _LANES // 2, pair_body,
                                    jnp.zeros((_LANES,), jnp.float32))
                out_v[pl.ds(off + eb, _LANES)] = res
                return 0

            lax.fori_loop(0, _CHUNK // _LANES, group_body, 0)

        bufs = ((u_0, v_0, sem_0), (u_1, v_1, sem_1),
                (u_2, v_2, sem_2), (u_3, v_3, sem_3))
        nbuf = len(bufs)
        for b, (ub, vb, sem) in enumerate(bufs):
            fire(b, ub, vb, sem)

        def quad_chunks(i, _):
            for b, (ub, vb, sem) in enumerate(bufs):
                ch = nbuf * i + b
                wait(ch, ub, vb, sem)
                compute(ch, ub, vb)

                @pl.when(i < nch // nbuf - 1)
                def _():
                    fire(ch + nbuf, ub, vb, sem)

            return 0

        lax.fori_loop(0, nch // nbuf, quad_chunks, 0)
        pltpu.sync_copy(out_v, out_hbm.at[pl.ds(base, epw)])

    return k


def kernel(feat, edge_index):
    n_nodes, d_feat = feat.shape
    n_edges = edge_index.shape[1]
    ei = edge_index.astype(jnp.int32)
    pad = (-n_edges) % (32 * _CHUNK * 4)
    src = jnp.pad(ei[0], (0, pad))
    dst = jnp.pad(ei[1], (0, pad))
    node_pad = (-n_nodes) % 256
    feat_bf = jnp.pad(feat.astype(jnp.bfloat16), ((0, node_pad), (0, 0)))
    feat_packed = jax.lax.bitcast_convert_type(
        feat_bf.reshape(n_nodes + node_pad, d_feat // 2, 2), jnp.int32)
    k = _make_sc_kernel(n_nodes + node_pad, d_feat, n_edges + pad)
    score = k(feat_packed, src, dst)
    return score[:n_edges].reshape(n_edges, 1)


# speedup vs baseline: 6.8705x; 1.2851x over previous
import functools

import jax
import jax.numpy as jnp
from jax import lax
from jax.experimental import pallas as pl
from jax.experimental.pallas import tpu as pltpu
from jax.experimental.pallas import tpu_sc as plsc

_LANES = 16
_CHUNK = 32
_NBUF = 3
_RB = 8

_DNUMS = lax.GatherDimensionNumbers(
    offset_dims=(), collapsed_slice_dims=(0,), start_index_map=(0,))


def _xlane(v, idx):
    return lax.gather(v, idx[:, None], dimension_numbers=_DNUMS,
                      slice_sizes=(1,),
                      mode=lax.GatherScatterMode.PROMISE_IN_BOUNDS)


@functools.lru_cache(maxsize=None)
def _make_sc_kernel(n_nodes, d_feat, n_edges_pad):
    info = plsc.get_sparse_core_info()
    nc, ns = info.num_cores, info.num_subcores
    nw = nc * ns
    assert n_edges_pad % (nw * _CHUNK) == 0
    epw = n_edges_pad // nw
    assert epw % 8 == 0
    nch = epw // _CHUNK
    d_half = d_feat // 2
    assert d_half % _LANES == 0
    assert n_nodes % _RB == 0
    rps = -(-(n_nodes // ns) // _RB) * _RB
    last_rows = n_nodes - rps * (ns - 1)
    assert 0 < last_rows <= rps and last_rows % _RB == 0

    mesh = plsc.VectorSubcoreMesh(core_axis_name="c", subcore_axis_name="s")

    @functools.partial(
        pl.kernel,
        mesh=mesh,
        out_type=jax.ShapeDtypeStruct((n_edges_pad,), jnp.float32),
        scratch_types=[
            pltpu.VMEM_SHARED((n_nodes, d_half), jnp.int32),
            pltpu.VMEM((epw,), jnp.int32),
            pltpu.VMEM((epw,), jnp.int32),
            pltpu.VMEM((_CHUNK, d_half), jnp.int32),
            pltpu.VMEM((_CHUNK, d_half), jnp.int32),
            pltpu.VMEM((_CHUNK, d_half), jnp.int32),
            pltpu.VMEM((_CHUNK, d_half), jnp.int32),
            pltpu.VMEM((_CHUNK, d_half), jnp.int32),
            pltpu.VMEM((_CHUNK, d_half), jnp.int32),
            pltpu.VMEM((_RB, d_feat), jnp.float32),
            pltpu.VMEM((_RB, d_feat), jnp.float32),
            pltpu.VMEM((epw,), jnp.float32),
            pltpu.SemaphoreType.DMA,
            pltpu.SemaphoreType.DMA,
            pltpu.SemaphoreType.DMA,
            pltpu.SemaphoreType.DMA,
            pltpu.SemaphoreType.DMA,
            pltpu.SemaphoreType.DMA,
            pltpu.SemaphoreType.DMA,
        ],
    )
    def k(feat_hbm, src_hbm, dst_hbm, out_hbm,
          table, src_v, dst_v, u_0, v_0, u_1, v_1, u_2, v_2,
          st_a, st_b, out_v, sem_0, sem_1, sem_2,
          sem_la, sem_lb, sem_pa, sem_pb):
        sid = lax.axis_index("s")
        wid = sid * nc + lax.axis_index("c")
        base = wid * epw
        pltpu.sync_copy(src_hbm.at[pl.ds(base, epw)], src_v)
        pltpu.sync_copy(dst_hbm.at[pl.ds(base, epw)], dst_v)
        lane = lax.iota(jnp.int32, _LANES)

        row0 = sid * rps
        nb = jnp.where(sid == ns - 1, last_rows // _RB, rps // _RB)
        half_bit = jnp.full((_LANES,), 0x8000, jnp.int32)
        hi_mask = jnp.full((_LANES,), -0x10000, jnp.int32)

        def ld_blk(blk, st, sem):
            return pltpu.make_async_copy(
                feat_hbm.at[pl.ds(row0 + blk * _RB, _RB)], st, sem)

        def wr_blk(blk, pk, sem):
            return pltpu.make_async_copy(
                pk.at[pl.ds(0, _RB)],
                table.at[pl.ds(row0 + blk * _RB, _RB)], sem)

        def pack_blk(st, pk):
            for r in range(_RB):
                for t in range(d_half // _LANES):
                    wl = st[r, pl.ds(t * _LANES, _LANES)]
                    wh = st[r, pl.ds(d_half + t * _LANES, _LANES)]
                    bl = lax.bitcast_convert_type(wl, jnp.int32)
                    bh = lax.bitcast_convert_type(wh, jnp.int32)
                    lo = lax.shift_right_logical(bl + half_bit, 16)
                    hi = (bh + half_bit) & hi_mask
                    pk[r, pl.ds(t * _LANES, _LANES)] = lo | hi

        ld_blk(0, st_a, sem_la).start()

        @pl.when(nb > 1)
        def _():
            ld_blk(1, st_b, sem_lb).start()

        def stage_pair(i, _):
            for p, (st, pk, sem_l, sem_p) in enumerate(
                    ((st_a, u_0, sem_la, sem_pa), (st_b, u_1, sem_lb, sem_pb))):
                blk = 2 * i + p

                @pl.when(blk < nb)
                def _():
                    ld_blk(blk, st, sem_l).wait()

                    @pl.when(blk >= 2)
                    def _():
                        wr_blk(blk - 2, pk, sem_p).wait()

                    pack_blk(st, pk)
                    wr_blk(blk, pk, sem_p).start()

                    @pl.when(blk + 2 < nb)
                    def _():
                        ld_blk(blk + 2, st, sem_l).start()

            return 0

        lax.fori_loop(0, (nb + 1) // 2, stage_pair, 0)

        @pl.when(nb >= 1)
        def _():
            wr_blk(0, u_0, sem_pa).wait()

        @pl.when(nb >= 2)
        def _():
            wr_blk(0, u_1, sem_pb).wait()

        plsc.subcore_barrier()

        def copies(ch, ub, vb, sem):
            off = pl.multiple_of(ch * _CHUNK, 8)
            cu = pltpu.make_async_copy(
                table.at[src_v.at[pl.ds(off, _CHUNK)]], ub, sem)
            cv = pltpu.make_async_copy(
                table.at[dst_v.at[pl.ds(off, _CHUNK)]], vb, sem)
            return cu, cv

        def fire(ch, ub, vb, sem):
            cu, cv = copies(ch, ub, vb, sem)
            cu.start()
            cv.start()

        def wait(ch, ub, vb, sem):
            cu, cv = copies(ch, ub, vb, sem)
            cu.wait()
            cv.wait()

        def compute(ch, ub, vb):
            off = pl.multiple_of(ch * _CHUNK, 8)

            def widen(wi):
                even = lax.bitcast_convert_type(wi << 16, jnp.float32)
                odd = lax.bitcast_convert_type(wi, jnp.float32)
                return even, odd

            def edge_dot(e):
                acc0 = jnp.zeros((_LANES,), jnp.float32)
                acc1 = jnp.zeros((_LANES,), jnp.float32)
                for t in range(d_half // _LANES):
                    uw = ub[e, pl.ds(t * _LANES, _LANES)]
                    vw = vb[e, pl.ds(t * _LANES, _LANES)]
                    u0, u1 = widen(uw)
                    v0, v1 = widen(vw)
                    acc0 = acc0 + u0 * v0
                    acc1 = acc1 + u1 * v1
                acc = acc0 + acc1
                for hop in (8, 4, 2, 1):
                    acc = acc + _xlane(acc, lane ^ hop)
                return acc

            def group_body(g, _):
                eb = g * _LANES

                def pair_body(j, res):
                    e = eb + 2 * j
                    res = jnp.where(lane == 2 * j, edge_dot(e), res)
                    res = jnp.where(lane == 2 * j + 1, edge_dot(e + 1), res)
                    return res

                res = lax.fori_loop(0, _LANES // 2, pair_body,
                                    jnp.zeros((_LANES,), jnp.float32))
                out_v[pl.ds(off + eb, _LANES)] = res
                return 0

            lax.fori_loop(0, _CHUNK // _LANES, group_body, 0)

        bufs = ((u_0, v_0, sem_0), (u_1, v_1, sem_1), (u_2, v_2, sem_2))
        for b, (ub, vb, sem) in enumerate(bufs):
            if b < nch:
                fire(b, ub, vb, sem)

        n_outer = -(-nch // _NBUF)

        def ring_chunks(i, _):
            for b, (ub, vb, sem) in enumerate(bufs):
                ch = _NBUF * i + b

                @pl.when(ch < nch)
                def _():
                    wait(ch, ub, vb, sem)
                    compute(ch, ub, vb)

                    @pl.when(ch + _NBUF < nch)
                    def _():
                        fire(ch + _NBUF, ub, vb, sem)

            return 0

        lax.fori_loop(0, n_outer, ring_chunks, 0)
        pltpu.sync_copy(out_v, out_hbm.at[pl.ds(base, epw)])

    return k


def kernel(feat, edge_index):
    n_nodes, d_feat = feat.shape
    n_edges = edge_index.shape[1]
    ei = edge_index.astype(jnp.int32)
    pad = (-n_edges) % (32 * _CHUNK)
    src = jnp.pad(ei[0], (0, pad))
    dst = jnp.pad(ei[1], (0, pad))
    k = _make_sc_kernel(n_nodes, d_feat, n_edges + pad)
    score = k(feat, src, dst)
    return score[:n_edges].reshape(n_edges, 1)
